# Initial kernel scaffold; baseline (speedup 1.0000x reference)
import jax
import jax.numpy as jnp
from jax.experimental import pallas as pl


def kernel(X, d, gamma, beta):
    raise NotImplementedError("write your pallas kernel here")


# speedup vs baseline: 5.3174x; 5.3174x over previous
import jax
import jax.numpy as jnp
from jax.experimental import pallas as pl
from jax.experimental.pallas import tpu as pltpu

_N_DOMAINS = 8
_EPS = 1e-5
_BT = 512


def _stats_kernel(d_ref, x_ref, sums_ref, sumsq_ref, cnt_ref):
    i = pl.program_id(0)
    x = x_ref[...]
    dvec = d_ref[0, 0, :]
    onehot = (
        dvec[:, None]
        == jax.lax.broadcasted_iota(jnp.int32, (dvec.shape[0], _N_DOMAINS), 1)
    ).astype(jnp.float32)
    s = jax.lax.dot(onehot.T, x, preferred_element_type=jnp.float32)
    sq = jax.lax.dot(onehot.T, x * x, preferred_element_type=jnp.float32)
    c = jnp.broadcast_to(jnp.sum(onehot, axis=0)[:, None], (_N_DOMAINS, 128))

    @pl.when(i == 0)
    def _():
        sums_ref[...] = s
        sumsq_ref[...] = sq
        cnt_ref[...] = c

    @pl.when(i != 0)
    def _():
        sums_ref[...] += s
        sumsq_ref[...] += sq
        cnt_ref[...] += c


def _apply_kernel(
    d_ref, sums_ref, sumsq_ref, cnt_ref, gamma_ref, beta_ref, x_ref,
    out_ref, scale_ref, shift_ref,
):
    i = pl.program_id(0)

    @pl.when(i == 0)
    def _():
        cnt = jnp.maximum(cnt_ref[:, 0:1], 1.0)
        mean = sums_ref[...] / cnt
        var = jnp.maximum(sumsq_ref[...] / cnt - mean * mean, 0.0)
        scale = gamma_ref[...] * jax.lax.rsqrt(var + _EPS)
        scale_ref[...] = scale
        shift_ref[...] = beta_ref[...] - mean * scale

    dvec = d_ref[0, 0, :]
    onehot = (
        dvec[:, None]
        == jax.lax.broadcasted_iota(jnp.int32, (dvec.shape[0], _N_DOMAINS), 1)
    ).astype(jnp.float32)
    sc = jax.lax.dot(onehot, scale_ref[...], preferred_element_type=jnp.float32)
    sh = jax.lax.dot(onehot, shift_ref[...], preferred_element_type=jnp.float32)
    out_ref[...] = x_ref[...] * sc + sh


def kernel(X, d, gamma, beta):
    nt, dm = X.shape
    nb = nt // _BT
    d_r = d.reshape(nb, 1, _BT)

    sums, sumsq, cnt = pl.pallas_call(
        _stats_kernel,
        grid=(nb,),
        in_specs=[
            pl.BlockSpec((1, 1, _BT), lambda i: (i, 0, 0)),
            pl.BlockSpec((_BT, dm), lambda i: (i, 0)),
        ],
        out_specs=[
            pl.BlockSpec((_N_DOMAINS, dm), lambda i: (0, 0)),
            pl.BlockSpec((_N_DOMAINS, dm), lambda i: (0, 0)),
            pl.BlockSpec((_N_DOMAINS, 128), lambda i: (0, 0)),
        ],
        out_shape=[
            jax.ShapeDtypeStruct((_N_DOMAINS, dm), jnp.float32),
            jax.ShapeDtypeStruct((_N_DOMAINS, dm), jnp.float32),
            jax.ShapeDtypeStruct((_N_DOMAINS, 128), jnp.float32),
        ],
    )(d_r, X)

    out = pl.pallas_call(
        _apply_kernel,
        grid=(nb,),
        in_specs=[
            pl.BlockSpec((1, 1, _BT), lambda i: (i, 0, 0)),
            pl.BlockSpec((_N_DOMAINS, dm), lambda i: (0, 0)),
            pl.BlockSpec((_N_DOMAINS, dm), lambda i: (0, 0)),
            pl.BlockSpec((_N_DOMAINS, 128), lambda i: (0, 0)),
            pl.BlockSpec((_N_DOMAINS, dm), lambda i: (0, 0)),
            pl.BlockSpec((_N_DOMAINS, dm), lambda i: (0, 0)),
            pl.BlockSpec((_BT, dm), lambda i: (i, 0)),
        ],
        out_specs=pl.BlockSpec((_BT, dm), lambda i: (i, 0)),
        out_shape=jax.ShapeDtypeStruct((nt, dm), jnp.float32),
        scratch_shapes=[
            pltpu.VMEM((_N_DOMAINS, dm), jnp.float32),
            pltpu.VMEM((_N_DOMAINS, dm), jnp.float32),
        ],
    )(d_r, sums, sumsq, cnt, gamma, beta, X)
    return out


# speedup vs baseline: 5.4135x; 1.0181x over previous
import jax
import jax.numpy as jnp
from jax.experimental import pallas as pl
from jax.experimental.pallas import tpu as pltpu

_N_DOMAINS = 8
_EPS = 1e-5
_BT = 512


def _bn_kernel(
    d_ref, x_ref, gamma_ref, beta_ref, out_ref,
    sums_ref, sumsq_ref, cnt_ref, scale_ref, shift_ref,
):
    p = pl.program_id(0)
    i = pl.program_id(1)
    dvec = d_ref[0, 0, :]
    onehot = (
        dvec[:, None]
        == jax.lax.broadcasted_iota(jnp.int32, (dvec.shape[0], _N_DOMAINS), 1)
    ).astype(jnp.float32)

    @pl.when(p == 0)
    def _stats():
        x = x_ref[...]
        s = jax.lax.dot(onehot.T, x, preferred_element_type=jnp.float32)
        sq = jax.lax.dot(onehot.T, x * x, preferred_element_type=jnp.float32)
        c = jnp.broadcast_to(jnp.sum(onehot, axis=0)[:, None], (_N_DOMAINS, 128))

        @pl.when(i == 0)
        def _():
            sums_ref[...] = s
            sumsq_ref[...] = sq
            cnt_ref[...] = c

        @pl.when(i != 0)
        def _():
            sums_ref[...] += s
            sumsq_ref[...] += sq
            cnt_ref[...] += c

    @pl.when(p == 1)
    def _apply():
        @pl.when(i == 0)
        def _():
            cnt = jnp.maximum(cnt_ref[:, 0:1], 1.0)
            mean = sums_ref[...] / cnt
            var = jnp.maximum(sumsq_ref[...] / cnt - mean * mean, 0.0)
            scale = gamma_ref[...] * jax.lax.rsqrt(var + _EPS)
            scale_ref[...] = scale
            shift_ref[...] = beta_ref[...] - mean * scale

        sc = jax.lax.dot(onehot, scale_ref[...], preferred_element_type=jnp.float32)
        sh = jax.lax.dot(onehot, shift_ref[...], preferred_element_type=jnp.float32)
        out_ref[...] = x_ref[...] * sc + sh


def kernel(X, d, gamma, beta):
    nt, dm = X.shape
    nb = nt // _BT
    d_r = d.reshape(nb, 1, _BT)

    out = pl.pallas_call(
        _bn_kernel,
        grid=(2, nb),
        in_specs=[
            pl.BlockSpec((1, 1, _BT), lambda p, i: (i, 0, 0)),
            pl.BlockSpec((_BT, dm), lambda p, i: (i, 0)),
            pl.BlockSpec((_N_DOMAINS, dm), lambda p, i: (0, 0)),
            pl.BlockSpec((_N_DOMAINS, dm), lambda p, i: (0, 0)),
        ],
        out_specs=pl.BlockSpec((_BT, dm), lambda p, i: (i * p, 0)),
        out_shape=jax.ShapeDtypeStruct((nt, dm), jnp.float32),
        scratch_shapes=[
            pltpu.VMEM((_N_DOMAINS, dm), jnp.float32),
            pltpu.VMEM((_N_DOMAINS, dm), jnp.float32),
            pltpu.VMEM((_N_DOMAINS, 128), jnp.float32),
            pltpu.VMEM((_N_DOMAINS, dm), jnp.float32),
            pltpu.VMEM((_N_DOMAINS, dm), jnp.float32),
        ],
    )(d_r, X, gamma, beta)
    return out
